# baseline (speedup 1.0000x reference)
import functools

import jax
import jax.numpy as jnp
from jax import lax
from jax.experimental import pallas as pl
from jax.experimental.pallas import tpu as pltpu
from jax.experimental.pallas import tpu_sc as plsc

VOCAB = 100000
MAX_LEN = 2048
EMB = 128
B, L = 4, 2048
N_ROWS = B * L

_info = plsc.get_sparse_core_info()
NC, NS = _info.num_cores, _info.num_subcores
NW = NC * NS
STRIPE = L // NW
ROWS_PER_L = L // STRIPE


def _body(ids_hbm, tok_hbm, pos_hbm, out_hbm, idx_v, rows_v, pos_v,
          sem_i, sem_p, sem_g, sem_o):
    wid = lax.axis_index("s") * NC + lax.axis_index("c")

    p_cp = pltpu.async_copy(pos_hbm.at[pl.ds(wid * STRIPE, STRIPE)],
                            pos_v, sem_p)

    i_cps = []
    for b in range(B):
        i_cps.append(pltpu.async_copy(
            ids_hbm.at[pl.ds(b * ROWS_PER_L + wid, 1)],
            idx_v.at[pl.ds(b, 1)], sem_i))
    for cp in i_cps:
        cp.wait()

    g_cps = []
    for b in range(B):
        g_cps.append(pltpu.async_copy(
            tok_hbm.at[idx_v.at[b]],
            rows_v.at[pl.ds(b * STRIPE, STRIPE)],
            sem_g.at[b]))
    p_cp.wait()

    o_cps = []
    for b in range(B):
        g_cps[b].wait()

        def add_row(i, _, b=b):
            for j in range(EMB // 16):
                s = pl.ds(j * 16, 16)
                rows_v[b * STRIPE + i, s] = (
                    rows_v[b * STRIPE + i, s] + pos_v[i, s])
            return 0

        lax.fori_loop(0, STRIPE, add_row, 0, unroll=2)
        o_cps.append(pltpu.async_copy(
            rows_v.at[pl.ds(b * STRIPE, STRIPE)],
            out_hbm.at[pl.ds(b * L + wid * STRIPE, STRIPE)],
            sem_o))
    for cp in o_cps:
        cp.wait()


@jax.jit
def _embed(ids2d, tok_table, pos_table):
    mesh = plsc.VectorSubcoreMesh(core_axis_name="c", subcore_axis_name="s")
    k = functools.partial(
        pl.kernel,
        mesh=mesh,
        out_type=jax.ShapeDtypeStruct((N_ROWS, EMB), jnp.float32),
        scratch_types=[
            pltpu.VMEM((B, STRIPE), jnp.int32),
            pltpu.VMEM((B * STRIPE, EMB), jnp.float32),
            pltpu.VMEM((STRIPE, EMB), jnp.float32),
            pltpu.SemaphoreType.DMA,
            pltpu.SemaphoreType.DMA,
            pltpu.SemaphoreType.DMA((B,)),
            pltpu.SemaphoreType.DMA,
        ],
    )(_body)
    return k(ids2d, tok_table, pos_table)


def kernel(inputs_ids, tok_table, pos_table):
    ids2d = inputs_ids.reshape(N_ROWS // STRIPE, STRIPE)
    out = _embed(ids2d, tok_table, pos_table)
    return out.reshape(B, L, EMB)


# speedup vs baseline: 1.1416x; 1.1416x over previous
import functools

import jax
import jax.numpy as jnp
from jax import lax
from jax.experimental import pallas as pl
from jax.experimental.pallas import tpu as pltpu
from jax.experimental.pallas import tpu_sc as plsc

VOCAB = 100000
MAX_LEN = 2048
EMB = 128
B, L = 4, 2048
N_ROWS = B * L

_info = plsc.get_sparse_core_info()
NC, NS = _info.num_cores, _info.num_subcores
NW = NC * NS
ROWS_PER_W = N_ROWS // NW
CHUNK = 32
N_CH = ROWS_PER_W // CHUNK


def _body(ids_hbm, tok_hbm, pos_hbm, out_hbm, idx_v, rows_v,
          sem_i, sem_p, sem_g, sem_o):
    wid = lax.axis_index("s") * NC + lax.axis_index("c")
    base = wid * ROWS_PER_W
    pos_base = lax.rem(base, MAX_LEN)

    idx_cp = pltpu.async_copy(
        ids_hbm.at[pl.ds(wid * N_CH, N_CH)], idx_v, sem_i)

    pos_cps = []
    for c in range(N_CH):
        pos_cps.append(pltpu.async_copy(
            pos_hbm.at[pl.ds(pos_base + c * CHUNK, CHUNK)],
            rows_v.at[pl.ds(c * CHUNK, CHUNK)],
            sem_p.at[c]))
    idx_cp.wait()

    g_cps = []
    for c in range(N_CH):
        pos_cps[c].wait()
        g_cps.append(pltpu.async_copy(
            tok_hbm.at[idx_v.at[c]],
            rows_v.at[pl.ds(c * CHUNK, CHUNK)],
            sem_g.at[c],
            add=True))

    o_cps = []
    for c in range(N_CH):
        g_cps[c].wait()
        o_cps.append(pltpu.async_copy(
            rows_v.at[pl.ds(c * CHUNK, CHUNK)],
            out_hbm.at[pl.ds(base + c * CHUNK, CHUNK)],
            sem_o.at[c]))
    for cp in o_cps:
        cp.wait()


@jax.jit
def _embed(ids2d, tok_table, pos_table):
    mesh = plsc.VectorSubcoreMesh(core_axis_name="c", subcore_axis_name="s")
    k = functools.partial(
        pl.kernel,
        mesh=mesh,
        out_type=jax.ShapeDtypeStruct((N_ROWS, EMB), jnp.float32),
        scratch_types=[
            pltpu.VMEM((N_CH, CHUNK), jnp.int32),
            pltpu.VMEM((ROWS_PER_W, EMB), jnp.float32),
            pltpu.SemaphoreType.DMA,
            pltpu.SemaphoreType.DMA((N_CH,)),
            pltpu.SemaphoreType.DMA((N_CH,)),
            pltpu.SemaphoreType.DMA((N_CH,)),
        ],
    )(_body)
    return k(ids2d, tok_table, pos_table)


def kernel(inputs_ids, tok_table, pos_table):
    ids2d = inputs_ids.reshape(N_ROWS // CHUNK, CHUNK)
    out = _embed(ids2d, tok_table, pos_table)
    return out.reshape(B, L, EMB)


# speedup vs baseline: 1.1631x; 1.0189x over previous
import functools

import jax
import jax.numpy as jnp
from jax import lax
from jax.experimental import pallas as pl
from jax.experimental.pallas import tpu as pltpu
from jax.experimental.pallas import tpu_sc as plsc

VOCAB = 100000
MAX_LEN = 2048
EMB = 128
B, L = 4, 2048
N_ROWS = B * L

_info = plsc.get_sparse_core_info()
NC, NS = _info.num_cores, _info.num_subcores
NW = NC * NS
ROWS_PER_W = N_ROWS // NW
CHUNK = 128
N_CH = ROWS_PER_W // CHUNK


def _body(ids_hbm, tok_hbm, pos_hbm, out_hbm, idx_v, rows_v,
          sem_i, sem_p, sem_g, sem_o):
    wid = lax.axis_index("s") * NC + lax.axis_index("c")
    base = wid * ROWS_PER_W
    pos_base = lax.rem(base, MAX_LEN)

    idx_cp = pltpu.async_copy(
        ids_hbm.at[pl.ds(wid * N_CH, N_CH)], idx_v, sem_i)

    pos_cps = []
    for c in range(N_CH):
        pos_cps.append(pltpu.async_copy(
            pos_hbm.at[pl.ds(pos_base + c * CHUNK, CHUNK)],
            rows_v.at[pl.ds(c * CHUNK, CHUNK)],
            sem_p.at[c]))
    idx_cp.wait()

    g_cps = []
    for c in range(N_CH):
        pos_cps[c].wait()
        g_cps.append(pltpu.async_copy(
            tok_hbm.at[idx_v.at[c]],
            rows_v.at[pl.ds(c * CHUNK, CHUNK)],
            sem_g.at[c],
            add=True))

    o_cps = []
    for c in range(N_CH):
        g_cps[c].wait()
        o_cps.append(pltpu.async_copy(
            rows_v.at[pl.ds(c * CHUNK, CHUNK)],
            out_hbm.at[pl.ds(base + c * CHUNK, CHUNK)],
            sem_o.at[c]))
    for cp in o_cps:
        cp.wait()


@jax.jit
def _embed(ids2d, tok_table, pos_table):
    mesh = plsc.VectorSubcoreMesh(core_axis_name="c", subcore_axis_name="s")
    k = functools.partial(
        pl.kernel,
        mesh=mesh,
        out_type=jax.ShapeDtypeStruct((N_ROWS, EMB), jnp.float32),
        scratch_types=[
            pltpu.VMEM((N_CH, CHUNK), jnp.int32),
            pltpu.VMEM((ROWS_PER_W, EMB), jnp.float32),
            pltpu.SemaphoreType.DMA,
            pltpu.SemaphoreType.DMA((N_CH,)),
            pltpu.SemaphoreType.DMA((N_CH,)),
            pltpu.SemaphoreType.DMA((N_CH,)),
        ],
    )(_body)
    return k(ids2d, tok_table, pos_table)


def kernel(inputs_ids, tok_table, pos_table):
    ids2d = inputs_ids.reshape(N_ROWS // CHUNK, CHUNK)
    out = _embed(ids2d, tok_table, pos_table)
    return out.reshape(B, L, EMB)
